# Initial kernel scaffold; baseline (speedup 1.0000x reference)
import jax
import jax.numpy as jnp
from jax.experimental import pallas as pl


def kernel(src, adj_dep, arc_tensor_in, arc_tensor_out, label_tensor_in, label_tensor_out, mask_in, mask_out, mask_loop, sent_mask, V_in, b_in, V_in_gate, b_in_gate, V_out, b_out, V_out_gate, b_out_gate, W_self_loop, W_self_loop_gate, ln_g, ln_b, dep_emb, dep_fc_W, dep_fc_b, Wf1, bf1, Wf2, bf2):
    raise NotImplementedError("write your pallas kernel here")


# speedup vs baseline: 11.9485x; 11.9485x over previous
import functools

import jax
import jax.numpy as jnp
from jax.experimental import pallas as pl

B, L, H, DEGR, NL_CON, NL_DEP = 8, 96, 512, 8, 64, 48
N = B * L


def _fused_body(x_ref, idx_in_ref, lab_in_ref, idx_out_ref, lab_out_ref,
                mask_in_ref, mask_out_ref, mask_loop_ref, sent_ref,
                V_in_ref, V_out_ref, Wsl_ref, Wslg_ref, Vg_in_ref, Vg_out_ref,
                b_in_ref, b_out_ref, bg_in_ref, bg_out_ref,
                adj_ref, dep_emb_ref, dep_fc_W_ref, dep_fc_b_ref,
                ln_g_ref, ln_b_ref,
                Wf1a_ref, Wf1b_ref, bf1_ref, Wf2a_ref, Wf2b_ref, bf2_ref,
                out_ref):
    f32 = jnp.float32
    x = x_ref[...]

    gv_in = jnp.dot(x, Vg_in_ref[...], preferred_element_type=f32)
    gv_out = jnp.dot(x, Vg_out_ref[...], preferred_element_type=f32)

    iota_n = jax.lax.broadcasted_iota(jnp.int32, (N, N), 1)
    iota_c = jax.lax.broadcasted_iota(jnp.int32, (N, NL_CON), 1)

    def build(idx_ref, lab_ref, mask_ref, gv, bg_ref):
        S = jnp.zeros((N, N), f32)
        T = jnp.zeros((N, NL_CON), f32)
        for d in range(DEGR):
            col = idx_ref[:, d:d + 1]
            lb = lab_ref[:, d:d + 1]
            oh = (iota_n == col).astype(f32)
            ohl = (iota_c == lb).astype(f32)
            g = jnp.dot(oh, gv, preferred_element_type=f32)
            bgd = jnp.dot(ohl, bg_ref[...], preferred_element_type=f32)
            p = jax.nn.sigmoid(g + bgd) * mask_ref[:, d:d + 1]
            S = S + p * oh
            T = T + p * ohl
        return S, T

    S_in, T_in = build(idx_in_ref, lab_in_ref, mask_in_ref, gv_in, bg_in_ref)
    S_out, T_out = build(idx_out_ref, lab_out_ref, mask_out_ref, gv_out, bg_out_ref)

    xV_i = jnp.dot(x, V_in_ref[...], preferred_element_type=f32)
    xV_o = jnp.dot(x, V_out_ref[...], preferred_element_type=f32)
    loop_gate = jax.nn.sigmoid(jnp.dot(x, Wslg_ref[...], preferred_element_type=f32))
    loop_eff = loop_gate * mask_loop_ref[...] * jnp.dot(x, Wsl_ref[...], preferred_element_type=f32)

    con0 = (jnp.dot(S_in, xV_i, preferred_element_type=f32)
            + jnp.dot(S_out, xV_o, preferred_element_type=f32)
            + jnp.dot(T_in, b_in_ref[...], preferred_element_type=f32)
            + jnp.dot(T_out, b_out_ref[...], preferred_element_type=f32)
            + loop_eff)
    con = jax.nn.relu(con0) * sent_ref[...]
    cr = con + x
    m = jnp.mean(cr, axis=-1, keepdims=True)
    v = jnp.mean((cr - m) ** 2, axis=-1, keepdims=True)
    con = (cr - m) * jax.lax.rsqrt(v + 1e-5) * ln_g_ref[...] + ln_b_ref[...]

    iota_d = jax.lax.broadcasted_iota(jnp.int32, (N, NL_DEP), 1)
    hist = jnp.zeros((N, NL_DEP), f32)
    for j in range(L):
        hist = hist + (iota_d == adj_ref[:, j:j + 1]).astype(f32)
    table = jnp.dot(dep_emb_ref[...], dep_fc_W_ref[...], preferred_element_type=f32)
    depsum = jnp.dot(hist, table, preferred_element_type=f32) + L * dep_fc_b_ref[...]
    dep = jax.nn.relu(x * depsum)

    g1 = jax.nn.sigmoid(jnp.dot(x, Wf1a_ref[...], preferred_element_type=f32)
                        + jnp.dot(con, Wf1b_ref[...], preferred_element_type=f32)
                        + bf1_ref[...])
    g2 = jax.nn.sigmoid(jnp.dot(x, Wf2a_ref[...], preferred_element_type=f32)
                        + jnp.dot(dep, Wf2b_ref[...], preferred_element_type=f32)
                        + bf2_ref[...])
    out_ref[...] = g1 * con + g2 * dep


@functools.partial(jax.jit, static_argnames=("interpret",))
def _run(args, interpret=False):
    return pl.pallas_call(
        _fused_body,
        out_shape=jax.ShapeDtypeStruct((N, H), jnp.float32),
        interpret=interpret,
    )(*args)


def kernel(src, adj_dep, arc_tensor_in, arc_tensor_out, label_tensor_in,
           label_tensor_out, mask_in, mask_out, mask_loop, sent_mask,
           V_in, b_in, V_in_gate, b_in_gate, V_out, b_out, V_out_gate,
           b_out_gate, W_self_loop, W_self_loop_gate, ln_g, ln_b,
           dep_emb, dep_fc_W, dep_fc_b, Wf1, bf1, Wf2, bf2,
           interpret=False):
    x = src.reshape(N, H)
    idx_in = (arc_tensor_in[0] * L + arc_tensor_in[1]).astype(jnp.int32).reshape(N, DEGR)
    idx_out = (arc_tensor_out[0] * L + arc_tensor_out[1]).astype(jnp.int32).reshape(N, DEGR)
    lab_in = label_tensor_in[0].astype(jnp.int32).reshape(N, DEGR)
    lab_out = label_tensor_out[0].astype(jnp.int32).reshape(N, DEGR)
    adj = adj_dep.astype(jnp.int32).reshape(N, L)

    args = (x, idx_in, lab_in, idx_out, lab_out,
            mask_in, mask_out, mask_loop, sent_mask.reshape(N, 1),
            V_in, V_out, W_self_loop, W_self_loop_gate, V_in_gate, V_out_gate,
            b_in, b_out, b_in_gate, b_out_gate,
            adj, dep_emb, dep_fc_W, dep_fc_b.reshape(1, H),
            ln_g.reshape(1, H), ln_b.reshape(1, H),
            Wf1[:H], Wf1[H:], bf1.reshape(1, H),
            Wf2[:H], Wf2[H:], bf2.reshape(1, H))
    out = _run(args, interpret=interpret)
    return out.reshape(B, L, H)
